# Initial kernel scaffold; baseline (speedup 1.0000x reference)
import jax
import jax.numpy as jnp
from jax.experimental import pallas as pl


def kernel(adj_indices, adj_values, user_id_emb, item_id_emb, user_visual_emb, user_acoustic_emb, user_textual_emb, visual_feat, acoustic_feat, textual_feat, W_v, W_a, W_t):
    raise NotImplementedError("write your pallas kernel here")


# speedup vs baseline: 2.7492x; 2.7492x over previous
import functools

import jax
import jax.numpy as jnp
from jax import lax
from jax.experimental import pallas as pl
from jax.experimental.pallas import tpu as pltpu
from jax.experimental.pallas import tpu_sc as plsc

N_USERS = 20000
N_ITEMS = 30000
N_NODES = N_USERS + N_ITEMS
EMB = 64
HALF = EMB // 2
N_EDGES = 800000
N_LAYERS = 2
N_SLABS = 6
TABLE_ROWS = N_SLABS * N_NODES

NC = 2
NS = 16
LANES = 16

CHUNK = 128
N_CHUNKS = N_EDGES // CHUNK
BASE_CHUNKS = N_CHUNKS // NS
EXTRA_CHUNKS = N_CHUNKS % NS
ROWS_PER_TEC = N_NODES // NS
ZROWS = 125
NZ = ROWS_PER_TEC // ZROWS


def _sc_spmm_kernel(row_hbm, col_hbm, val_hbm, table_hbm, out_hbm,
                    colv, rowv, valv, vals, rowsv, zbuf, accum, sem):
    c = lax.axis_index("c")
    s = lax.axis_index("s")

    zero16 = jnp.zeros((LANES,), jnp.float32)

    def fill_zero(e, carry):
        zbuf[e, pl.ds(0, LANES)] = zero16
        zbuf[e, pl.ds(LANES, LANES)] = zero16
        return carry

    lax.fori_loop(0, ZROWS, fill_zero, 0)

    n_chunks_me = BASE_CHUNKS + jnp.where(s < EXTRA_CHUNKS, 1, 0)
    my_rows = s * ROWS_PER_TEC

    def slab_body(j, carry):
        slab_off = c * (3 * N_NODES) + j * N_NODES

        def zero_body(i, carry2):
            pltpu.sync_copy(zbuf, accum.at[pl.ds(my_rows + i * ZROWS, ZROWS)])
            return carry2

        lax.fori_loop(0, NZ, zero_body, 0)
        plsc.subcore_barrier()

        def chunk_body(k, carry2):
            eb = (s + k * NS) * CHUNK
            pltpu.sync_copy(col_hbm.at[pl.ds(eb, CHUNK)], colv)
            pltpu.sync_copy(row_hbm.at[pl.ds(eb, CHUNK)], rowv)
            pltpu.sync_copy(val_hbm.at[pl.ds(eb, CHUNK)], valv)
            for t in range(CHUNK // LANES):
                sl = pl.ds(t * LANES, LANES)
                colv[sl] = colv[sl] + slab_off
            pltpu.async_copy(table_hbm.at[colv], rowsv, sem).wait()

            def scale_body(g, carry3):
                vv = valv[pl.ds(g * LANES, LANES)]
                for i in range(LANES):
                    e = g * LANES + i
                    vs = vv[i]
                    a = rowsv[e, pl.ds(0, LANES)]
                    rowsv[e, pl.ds(0, LANES)] = a * vs
                    b = rowsv[e, pl.ds(LANES, LANES)]
                    rowsv[e, pl.ds(LANES, LANES)] = b * vs
                return carry3

            lax.fori_loop(0, CHUNK // LANES, scale_body, 0)
            pltpu.sync_copy(rowsv, accum.at[rowv], add=True)
            return carry2

        lax.fori_loop(0, n_chunks_me, chunk_body, 0)
        plsc.subcore_barrier()

        def flush_body(i, carry2):
            r0 = my_rows + i * ZROWS
            pltpu.sync_copy(accum.at[pl.ds(r0, ZROWS)],
                            out_hbm.at[pl.ds(slab_off + r0, ZROWS)])
            return carry2

        lax.fori_loop(0, NZ, flush_body, 0)
        plsc.subcore_barrier()
        return carry

    lax.fori_loop(0, 3, slab_body, 0)


def _make_sc_spmm():
    mesh = plsc.VectorSubcoreMesh(core_axis_name="c", subcore_axis_name="s")
    return functools.partial(
        pl.kernel,
        mesh=mesh,
        compiler_params=pltpu.CompilerParams(use_tc_tiling_on_sc=False),
        out_type=jax.ShapeDtypeStruct((TABLE_ROWS, HALF), jnp.float32),
        scratch_types=[
            pltpu.VMEM((CHUNK,), jnp.int32),
            pltpu.VMEM((CHUNK,), jnp.int32),
            pltpu.VMEM((CHUNK,), jnp.float32),
            pltpu.SMEM((8,), jnp.float32),
            pltpu.VMEM((CHUNK, HALF), jnp.float32),
            pltpu.VMEM((ZROWS, HALF), jnp.float32),
            pltpu.VMEM_SHARED((N_NODES, HALF), jnp.float32),
            pltpu.SemaphoreType.DMA,
        ],
    )(_sc_spmm_kernel)


ROW_BLK = 2000


def _tc_mid_body(s_ref, w_ref, o_ref):
    for m in range(3):
        h = jnp.concatenate([s_ref[m], s_ref[3 + m]], axis=1)
        y = jnp.dot(h, w_ref[m].T, preferred_element_type=jnp.float32)
        y = jnp.where(y >= 0, y, 0.2 * y)
        o_ref[m] = y[:, :HALF]
        o_ref[3 + m] = y[:, HALF:]


def _tc_fin_body(s_ref, w_ref, u_ref, o_ref):
    acc = u_ref[...]
    for m in range(3):
        h = jnp.concatenate([s_ref[m], s_ref[3 + m]], axis=1)
        y = jnp.dot(h, w_ref[m].T, preferred_element_type=jnp.float32)
        acc = acc + jnp.where(y >= 0, y, 0.2 * y)
    o_ref[...] = acc


def _tc_mid(s6, w):
    grid = (N_NODES // ROW_BLK,)
    return pl.pallas_call(
        _tc_mid_body,
        grid=grid,
        in_specs=[
            pl.BlockSpec((N_SLABS, ROW_BLK, HALF), lambda i: (0, i, 0)),
            pl.BlockSpec((3, EMB, EMB), lambda i: (0, 0, 0)),
        ],
        out_specs=pl.BlockSpec((N_SLABS, ROW_BLK, HALF), lambda i: (0, i, 0)),
        out_shape=jax.ShapeDtypeStruct((N_SLABS, N_NODES, HALF), jnp.float32),
    )(s6, w)


def _tc_fin(s6, w, uid):
    grid = (N_NODES // ROW_BLK,)
    return pl.pallas_call(
        _tc_fin_body,
        grid=grid,
        in_specs=[
            pl.BlockSpec((N_SLABS, ROW_BLK, HALF), lambda i: (0, i, 0)),
            pl.BlockSpec((3, EMB, EMB), lambda i: (0, 0, 0)),
            pl.BlockSpec((ROW_BLK, EMB), lambda i: (i, 0)),
        ],
        out_specs=pl.BlockSpec((ROW_BLK, EMB), lambda i: (i, 0)),
        out_shape=jax.ShapeDtypeStruct((N_NODES, EMB), jnp.float32),
    )(s6, w, uid)


def kernel(adj_indices, adj_values, user_id_emb, item_id_emb,
           user_visual_emb, user_acoustic_emb, user_textual_emb,
           visual_feat, acoustic_feat, textual_feat, W_v, W_a, W_t):
    row = adj_indices[0]
    col = adj_indices[1]

    uid = jnp.concatenate([user_id_emb, item_id_emb], axis=0)
    vis = jnp.concatenate([user_visual_emb, visual_feat], axis=0)
    aco = jnp.concatenate([user_acoustic_emb, acoustic_feat], axis=0)
    tex = jnp.concatenate([user_textual_emb, textual_feat], axis=0)

    table = jnp.concatenate(
        [vis[:, :HALF], aco[:, :HALF], tex[:, :HALF],
         vis[:, HALF:], aco[:, HALF:], tex[:, HALF:]], axis=0)

    sc_spmm = _make_sc_spmm()

    fused = None
    for l in range(N_LAYERS):
        w = jnp.stack([W_v[l], W_a[l], W_t[l]])
        spmm = sc_spmm(row, col, adj_values, table)
        s6 = spmm.reshape(N_SLABS, N_NODES, HALF)
        if l + 1 < N_LAYERS:
            table = _tc_mid(s6, w).reshape(TABLE_ROWS, HALF)
        else:
            fused = _tc_fin(s6, w, uid)

    return (fused[:N_USERS], fused[N_USERS:])
